# Initial kernel scaffold; baseline (speedup 1.0000x reference)
import jax
import jax.numpy as jnp
from jax.experimental import pallas as pl


def kernel(X, STE, W7, b7, W8, b8, W9, b9, W10, b10, W11, b11, node_emb):
    raise NotImplementedError("write your pallas kernel here")


# speedup vs baseline: 18.3626x; 18.3626x over previous
import jax
import jax.numpy as jnp
from jax import lax
from jax.experimental import pallas as pl

K_HEADS = 8
D_HEAD = 16
TOPK = 10
MEM = 30
MEM_PAD = 32
QKV_W = 3 * D_HEAD
NEG = -1e30

NN_DIMS = (((1,), (0,)), ((), ()))
NT_DIMS = (((1,), (1,)), ((), ()))
TN_DIMS = (((0,), (0,)), ((), ()))


def _body(x_ref, ste_ref, wxt_ref, wst_ref, bqkv_ref, el_ref,
          w10t_ref, b10_ref, w11t_ref, b11_ref, out_ref):
    N = x_ref.shape[2]
    X = x_ref[0, 0]
    STE = ste_ref[0, 0]
    f32 = jnp.float32

    def dg(a, b, dims):
        return lax.dot_general(a, b, dims, preferred_element_type=f32)

    qkvt = dg(wxt_ref[...], X, NT_DIMS) + dg(wst_ref[...], STE, NT_DIMS)
    qkvt = jnp.maximum(qkvt + bqkv_ref[...], 0.0)

    L = dg(el_ref[...], qkvt, NN_DIMS)

    rowvalid = lax.broadcasted_iota(jnp.int32, (MEM_PAD, N), 0) < MEM

    R = TOPK * MEM_PAD
    ri = lax.broadcasted_iota(jnp.int32, (R, R), 0) % MEM_PAD
    ci = lax.broadcasted_iota(jnp.int32, (R, R), 1) % MEM_PAD
    valid = (ri == ci) & (ri < MEM) & (ci < MEM)
    maskadd = jnp.where(valid, 0.0, NEG)

    x1_parts = []
    for h in range(K_HEADS):
        Lh = L[MEM_PAD * h: MEM_PAD * (h + 1), :]
        ohs = []
        for _ in range(TOPK):
            m = jnp.max(Lh, axis=1, keepdims=True)
            eq = Lh >= m
            ohs.append(jnp.where(eq & rowvalid, 1.0, 0.0))
            Lh = jnp.where(eq, NEG, Lh)
        S = jnp.concatenate(ohs, axis=0)

        qkv_h = qkvt[QKV_W * h: QKV_W * (h + 1), :]
        selT = dg(qkv_h, S, NT_DIMS)
        selQT = selT[:D_HEAD, :]
        selKT = selT[D_HEAD:2 * D_HEAD, :]
        selVT = selT[2 * D_HEAD:, :]

        A = dg(selQT, selKT, TN_DIMS)
        Am = A * 0.25 + maskadd
        mx = jnp.max(Am, axis=1, keepdims=True)
        E = jnp.exp(Am - mx)
        P = E / jnp.sum(E, axis=1, keepdims=True)
        nn = dg(P, selVT, NT_DIMS)

        dict_h = dg(S, nn, TN_DIMS)
        counts = jnp.sum(S, axis=0)[:, None] + 1e-14
        x1_parts.append(dict_h / counts)

    X1 = jnp.concatenate(x1_parts, axis=1)
    Hd = jnp.maximum(dg(X1, w10t_ref[...], NN_DIMS) + b10_ref[...], 0.0)
    out_ref[0, 0] = dg(Hd, w11t_ref[...], NN_DIMS) + b11_ref[...]


def kernel(X, STE, W7, b7, W8, b8, W9, b9, W10, b10, W11, b11, node_emb):
    B, T, N, D = X.shape
    f32 = jnp.float32

    def head_blocks(W):
        return (W[:, :D].reshape(K_HEADS, D_HEAD, D),
                W[:, D:].reshape(K_HEADS, D_HEAD, D))

    w7x, w7s = head_blocks(W7)
    w8x, w8s = head_blocks(W8)
    w9x, w9s = head_blocks(W9)
    WxT = jnp.concatenate([w7x, w8x, w9x], axis=1).reshape(3 * D, D)
    WsT = jnp.concatenate([w7s, w8s, w9s], axis=1).reshape(3 * D, D)
    bqkv = jnp.concatenate(
        [b7.reshape(K_HEADS, D_HEAD), b8.reshape(K_HEADS, D_HEAD),
         b9.reshape(K_HEADS, D_HEAD)], axis=1).reshape(3 * D, 1)

    blk = jnp.concatenate([node_emb, jnp.zeros((MEM, D_HEAD), f32)], axis=1)
    blk = jnp.concatenate(
        [blk, jnp.zeros((MEM_PAD - MEM, QKV_W), f32)], axis=0)
    EL = jnp.kron(jnp.eye(K_HEADS, dtype=f32), blk)

    out = pl.pallas_call(
        _body,
        grid=(B, T),
        in_specs=[
            pl.BlockSpec((1, 1, N, D), lambda b, t: (b, t, 0, 0)),
            pl.BlockSpec((1, 1, N, D), lambda b, t: (b, t, 0, 0)),
            pl.BlockSpec((3 * D, D), lambda b, t: (0, 0)),
            pl.BlockSpec((3 * D, D), lambda b, t: (0, 0)),
            pl.BlockSpec((3 * D, 1), lambda b, t: (0, 0)),
            pl.BlockSpec((2 * K_HEADS * D_HEAD, 3 * D), lambda b, t: (0, 0)),
            pl.BlockSpec((D, D), lambda b, t: (0, 0)),
            pl.BlockSpec((1, D), lambda b, t: (0, 0)),
            pl.BlockSpec((D, D), lambda b, t: (0, 0)),
            pl.BlockSpec((1, D), lambda b, t: (0, 0)),
        ],
        out_specs=pl.BlockSpec((1, 1, N, D), lambda b, t: (b, t, 0, 0)),
        out_shape=jax.ShapeDtypeStruct((B, T, N, D), f32),
    )(X, STE, WxT, WsT, bqkv, EL, W10.T, b10.reshape(1, D),
      W11.T, b11.reshape(1, D))
    return out


# speedup vs baseline: 19.3207x; 1.0522x over previous
import jax
import jax.numpy as jnp
from jax import lax
from jax.experimental import pallas as pl

K_HEADS = 8
D_HEAD = 16
TOPK = 10
MEM = 30
MEM_PAD = 32
QKV_W = 3 * D_HEAD
NEG = -1e30

NN_DIMS = (((1,), (0,)), ((), ()))
NT_DIMS = (((1,), (1,)), ((), ()))
TN_DIMS = (((0,), (0,)), ((), ()))


def _body(x_ref, ste_ref, wxt_ref, wst_ref, bqkv_ref, el_ref,
          w10t_ref, b10_ref, w11t_ref, b11_ref, out_ref):
    N = x_ref.shape[2]
    X = x_ref[0, 0]
    STE = ste_ref[0, 0]
    f32 = jnp.float32

    def dg(a, b, dims):
        return lax.dot_general(a, b, dims, preferred_element_type=f32)

    qkvt = dg(wxt_ref[...], X, NT_DIMS) + dg(wst_ref[...], STE, NT_DIMS)
    qkvt = jnp.maximum(qkvt + bqkv_ref[...], 0.0)

    L = dg(el_ref[...], qkvt, NN_DIMS)

    rowvalid = lax.broadcasted_iota(jnp.int32, (MEM_PAD, N), 0) < MEM

    R = TOPK * MEM_PAD
    ri = lax.broadcasted_iota(jnp.int32, (R, R), 0) % MEM_PAD
    ci = lax.broadcasted_iota(jnp.int32, (R, R), 1) % MEM_PAD
    valid = (ri == ci) & (ri < MEM) & (ci < MEM)
    maskadd = jnp.where(valid, 0.0, NEG)
    bf16 = jnp.bfloat16
    ones_col = jnp.ones((R, D_HEAD), bf16)

    x1_parts = []
    for h in range(K_HEADS):
        Lh = L[MEM_PAD * h: MEM_PAD * (h + 1), :]
        ohs = []
        for _ in range(TOPK):
            m = jnp.max(Lh, axis=1, keepdims=True)
            eq = Lh >= m
            ohs.append((eq & rowvalid).astype(bf16))
            Lh = jnp.where(eq, NEG, Lh)
        S = jnp.concatenate(ohs, axis=0)

        qkv_h = qkvt[QKV_W * h: QKV_W * (h + 1), :]
        selT = dg(qkv_h.astype(bf16), S, NT_DIMS)
        selQT = selT[:D_HEAD, :]
        selKT = selT[D_HEAD:2 * D_HEAD, :]
        selVT = selT[2 * D_HEAD:, :]

        A = dg(selQT, selKT, TN_DIMS)
        E = jnp.exp(A * 0.25 + maskadd)
        P = E * (1.0 / (jnp.sum(E, axis=1, keepdims=True) + 1e-30))
        nn = dg(P, selVT, NT_DIMS)

        nn_aug = jnp.concatenate([nn.astype(bf16), ones_col], axis=1)
        dict_aug = dg(S, nn_aug, TN_DIMS)
        rc = 1.0 / (dict_aug[:, D_HEAD:D_HEAD + 1] + 1e-14)
        x1_parts.append(dict_aug[:, :D_HEAD] * rc)

    X1 = jnp.concatenate(x1_parts, axis=1)
    Hd = jnp.maximum(dg(X1, w10t_ref[...], NN_DIMS) + b10_ref[...], 0.0)
    out_ref[0, 0] = dg(Hd, w11t_ref[...], NN_DIMS) + b11_ref[...]


def kernel(X, STE, W7, b7, W8, b8, W9, b9, W10, b10, W11, b11, node_emb):
    B, T, N, D = X.shape
    f32 = jnp.float32

    def head_blocks(W):
        return (W[:, :D].reshape(K_HEADS, D_HEAD, D),
                W[:, D:].reshape(K_HEADS, D_HEAD, D))

    w7x, w7s = head_blocks(W7)
    w8x, w8s = head_blocks(W8)
    w9x, w9s = head_blocks(W9)
    WxT = jnp.concatenate([w7x, w8x, w9x], axis=1).reshape(3 * D, D)
    WsT = jnp.concatenate([w7s, w8s, w9s], axis=1).reshape(3 * D, D)
    bqkv = jnp.concatenate(
        [b7.reshape(K_HEADS, D_HEAD), b8.reshape(K_HEADS, D_HEAD),
         b9.reshape(K_HEADS, D_HEAD)], axis=1).reshape(3 * D, 1)

    blk = jnp.concatenate([node_emb, jnp.zeros((MEM, D_HEAD), f32)], axis=1)
    blk = jnp.concatenate(
        [blk, jnp.zeros((MEM_PAD - MEM, QKV_W), f32)], axis=0)
    EL = jnp.kron(jnp.eye(K_HEADS, dtype=f32), blk)

    out = pl.pallas_call(
        _body,
        grid=(B, T),
        in_specs=[
            pl.BlockSpec((1, 1, N, D), lambda b, t: (b, t, 0, 0)),
            pl.BlockSpec((1, 1, N, D), lambda b, t: (b, t, 0, 0)),
            pl.BlockSpec((3 * D, D), lambda b, t: (0, 0)),
            pl.BlockSpec((3 * D, D), lambda b, t: (0, 0)),
            pl.BlockSpec((3 * D, 1), lambda b, t: (0, 0)),
            pl.BlockSpec((2 * K_HEADS * D_HEAD, 3 * D), lambda b, t: (0, 0)),
            pl.BlockSpec((D, D), lambda b, t: (0, 0)),
            pl.BlockSpec((1, D), lambda b, t: (0, 0)),
            pl.BlockSpec((D, D), lambda b, t: (0, 0)),
            pl.BlockSpec((1, D), lambda b, t: (0, 0)),
        ],
        out_specs=pl.BlockSpec((1, 1, N, D), lambda b, t: (b, t, 0, 0)),
        out_shape=jax.ShapeDtypeStruct((B, T, N, D), f32),
    )(X, STE, WxT, WsT, bqkv, EL, W10.T, b10.reshape(1, D),
      W11.T, b11.reshape(1, D))
    return out


# speedup vs baseline: 22.1098x; 1.1444x over previous
import numpy as np
import jax
import jax.numpy as jnp
from jax import lax
from jax.experimental import pallas as pl

K_HEADS = 8
D_HEAD = 16
TOPK = 10
MEM = 30
MEM_PAD = 32
NEG = -1e30
QK = 2 * D_HEAD
NQK = K_HEADS * QK
R = TOPK * MEM_PAD

NN_DIMS = (((1,), (0,)), ((), ()))
NT_DIMS = (((1,), (1,)), ((), ()))
TN_DIMS = (((0,), (0,)), ((), ()))


def _place_matrix():
    p = np.zeros((K_HEADS, QK, 2 * K_HEADS * D_HEAD), np.float32)
    for h in range(K_HEADS):
        for j in range(D_HEAD):
            p[h, j, D_HEAD * h + j] = 1.0
            p[h, D_HEAD + j, K_HEADS * D_HEAD + D_HEAD * h + j] = 1.0
    return p


_PLACE = _place_matrix()


def _body(x_ref, ste_ref, wxt_ref, wst_ref, bqkv_ref, elqk_ref, place_ref,
          w10t_ref, b10_ref, w11t_ref, b11_ref, out_ref):
    N = x_ref.shape[2]
    X = x_ref[0, 0]
    STE = ste_ref[0, 0]
    f32 = jnp.float32
    bf16 = jnp.bfloat16

    qkvt = lax.dot_general(wxt_ref[...], X, NT_DIMS, preferred_element_type=f32)
    qkvt += lax.dot_general(wst_ref[...], STE, NT_DIMS, preferred_element_type=f32)
    qkvt = jnp.maximum(qkvt + bqkv_ref[...], 0.0)

    L = lax.dot_general(elqk_ref[...], qkvt[:NQK, :], NN_DIMS,
                        preferred_element_type=f32)

    padbump = jnp.where(
        lax.broadcasted_iota(jnp.int32, (MEM_PAD, 1), 0) < MEM, 0.0, 1e30)

    ri = lax.broadcasted_iota(jnp.int32, (R, R), 0) % MEM_PAD
    ci = lax.broadcasted_iota(jnp.int32, (R, R), 1) % MEM_PAD
    valid = (ri == ci) & (ri < MEM) & (ci < MEM)
    maskadd = jnp.where(valid, 0.0, NEG)
    ones_col = jnp.ones((R, D_HEAD), bf16)

    ohs = []
    for h in range(K_HEADS):
        Lh = L[QK * h: QK * h + MEM_PAD, :]
        for _ in range(TOPK):
            m = jnp.max(Lh, axis=1, keepdims=True) + padbump
            eq = Lh >= m
            ohs.append(eq.astype(bf16))
            Lh = jnp.where(eq, NEG, Lh)
    S_cat = jnp.concatenate(ohs, axis=0)

    nnblks = []
    for h in range(K_HEADS):
        S_h = S_cat[R * h: R * (h + 1), :]
        qk_b = qkvt[QK * h: QK * (h + 1), :].astype(bf16)
        v_b = qkvt[NQK + D_HEAD * h: NQK + D_HEAD * (h + 1), :].astype(bf16)
        selQK = lax.dot_general(qk_b, S_h, NT_DIMS, preferred_element_type=f32)
        selV = lax.dot_general(v_b, S_h, NT_DIMS, preferred_element_type=f32)

        A = lax.dot_general(selQK[:D_HEAD, :], selQK[D_HEAD:, :], TN_DIMS,
                            preferred_element_type=f32)
        E = jnp.exp(A * 0.25 + maskadd)
        P = E * (1.0 / (jnp.sum(E, axis=1, keepdims=True) + 1e-30))
        nn = lax.dot_general(P, selV, NT_DIMS,
                             preferred_element_type=f32)
        nn_aug = jnp.concatenate([nn.astype(bf16), ones_col], axis=1)
        nnblks.append(lax.dot_general(nn_aug, place_ref[h], NN_DIMS,
                                      preferred_element_type=f32).astype(bf16))
    NNblk = jnp.concatenate(nnblks, axis=0)

    X1aug = lax.dot_general(S_cat, NNblk, TN_DIMS, preferred_element_type=f32)
    D = K_HEADS * D_HEAD
    X1 = X1aug[:, :D] * (1.0 / (X1aug[:, D:] + 1e-14))

    Hd = lax.dot_general(X1, w10t_ref[...], NN_DIMS, preferred_element_type=f32)
    Hd = jnp.maximum(Hd + b10_ref[...], 0.0)
    out = lax.dot_general(Hd, w11t_ref[...], NN_DIMS, preferred_element_type=f32)
    out_ref[0, 0] = out + b11_ref[...]


def kernel(X, STE, W7, b7, W8, b8, W9, b9, W10, b10, W11, b11, node_emb):
    B, T, N, D = X.shape
    f32 = jnp.float32

    def head_blocks(W):
        return (W[:, :D].reshape(K_HEADS, D_HEAD, D),
                W[:, D:].reshape(K_HEADS, D_HEAD, D))

    w7x, w7s = head_blocks(W7)
    w8x, w8s = head_blocks(W8)
    w9x, w9s = head_blocks(W9)
    WxT = jnp.concatenate(
        [jnp.concatenate([w7x, w8x], axis=1).reshape(NQK, D),
         w9x.reshape(K_HEADS * D_HEAD, D)], axis=0)
    WsT = jnp.concatenate(
        [jnp.concatenate([w7s, w8s], axis=1).reshape(NQK, D),
         w9s.reshape(K_HEADS * D_HEAD, D)], axis=0)
    bqkv = jnp.concatenate(
        [jnp.concatenate([b7.reshape(K_HEADS, D_HEAD),
                          b8.reshape(K_HEADS, D_HEAD)], axis=1).reshape(NQK),
         b9], axis=0).reshape(3 * D, 1)

    embpad = jnp.concatenate(
        [node_emb, jnp.zeros((MEM_PAD - MEM, QK), f32)], axis=0)
    ELqk = jnp.kron(jnp.eye(K_HEADS, dtype=f32), embpad)
    place = jnp.asarray(_PLACE, jnp.bfloat16)

    out = pl.pallas_call(
        _body,
        grid=(B, T),
        in_specs=[
            pl.BlockSpec((1, 1, N, D), lambda b, t: (b, t, 0, 0)),
            pl.BlockSpec((1, 1, N, D), lambda b, t: (b, t, 0, 0)),
            pl.BlockSpec((3 * D, D), lambda b, t: (0, 0)),
            pl.BlockSpec((3 * D, D), lambda b, t: (0, 0)),
            pl.BlockSpec((3 * D, 1), lambda b, t: (0, 0)),
            pl.BlockSpec((NQK, NQK), lambda b, t: (0, 0)),
            pl.BlockSpec((K_HEADS, QK, NQK), lambda b, t: (0, 0, 0)),
            pl.BlockSpec((D, D), lambda b, t: (0, 0)),
            pl.BlockSpec((1, D), lambda b, t: (0, 0)),
            pl.BlockSpec((D, D), lambda b, t: (0, 0)),
            pl.BlockSpec((1, D), lambda b, t: (0, 0)),
        ],
        out_specs=pl.BlockSpec((1, 1, N, D), lambda b, t: (b, t, 0, 0)),
        out_shape=jax.ShapeDtypeStruct((B, T, N, D), f32),
    )(X, STE, WxT, WsT, bqkv, ELqk, place, W10.T, b10.reshape(1, D),
      W11.T, b11.reshape(1, D))
    return out


# speedup vs baseline: 22.4796x; 1.0167x over previous
import numpy as np
import jax
import jax.numpy as jnp
from jax import lax
from jax.experimental import pallas as pl

K_HEADS = 8
D_HEAD = 16
TOPK = 10
MEM = 30
MEM_PAD = 32
NEG = -1e30
QK = 2 * D_HEAD
NQK = K_HEADS * QK
R = TOPK * MEM_PAD

NN_DIMS = (((1,), (0,)), ((), ()))
NT_DIMS = (((1,), (1,)), ((), ()))
TN_DIMS = (((0,), (0,)), ((), ()))


def _place_matrix():
    p = np.zeros((K_HEADS, QK, 2 * K_HEADS * D_HEAD), np.float32)
    for h in range(K_HEADS):
        for j in range(D_HEAD):
            p[h, j, D_HEAD * h + j] = 1.0
            p[h, D_HEAD + j, K_HEADS * D_HEAD + D_HEAD * h + j] = 1.0
    return p


_PLACE = _place_matrix()


def _body(x_ref, ste_ref, wxt_ref, wst_ref, bqkv_ref, elqk_ref, place_ref,
          w10t_ref, b10_ref, w11t_ref, b11_ref, out_ref):
    N = x_ref.shape[2]
    X = x_ref[0, 0]
    STE = ste_ref[0, 0]
    f32 = jnp.float32
    bf16 = jnp.bfloat16

    qkvt = lax.dot_general(wxt_ref[...], X, NT_DIMS, preferred_element_type=f32)
    qkvt += lax.dot_general(wst_ref[...], STE, NT_DIMS, preferred_element_type=f32)
    qkvt = jnp.maximum(qkvt + bqkv_ref[...], 0.0)

    L = lax.dot_general(elqk_ref[...], qkvt[:NQK, :], NN_DIMS,
                        preferred_element_type=f32)

    padbump = jnp.where(
        lax.broadcasted_iota(jnp.int32, (MEM_PAD, 1), 0) < MEM, 0.0, 1e30)

    ri = lax.broadcasted_iota(jnp.int32, (R, R), 0) % MEM_PAD
    ci = lax.broadcasted_iota(jnp.int32, (R, R), 1) % MEM_PAD
    valid = (ri == ci) & (ri < MEM) & (ci < MEM)
    maskadd = jnp.where(valid, 0.0, NEG)
    ones_col = jnp.ones((R, D_HEAD), bf16)

    ohs = []
    for h in range(K_HEADS):
        Lh = L[QK * h: QK * h + MEM_PAD, :]
        for _ in range(TOPK):
            m = jnp.max(Lh, axis=1, keepdims=True) + padbump
            eq = Lh >= m
            ohs.append(eq.astype(bf16))
            Lh = jnp.where(eq, NEG, Lh)
    S_cat = jnp.concatenate(ohs, axis=0)

    nnblks = []
    for h in range(K_HEADS):
        S_h = S_cat[R * h: R * (h + 1), :]
        qk_b = qkvt[QK * h: QK * (h + 1), :].astype(bf16)
        v_b = qkvt[NQK + D_HEAD * h: NQK + D_HEAD * (h + 1), :].astype(bf16)
        selQK = lax.dot_general(qk_b, S_h, NT_DIMS,
                                preferred_element_type=f32).astype(bf16)
        selV = lax.dot_general(v_b, S_h, NT_DIMS,
                               preferred_element_type=f32).astype(bf16)

        A = lax.dot_general(selQK[:D_HEAD, :], selQK[D_HEAD:, :], TN_DIMS,
                            preferred_element_type=f32)
        E = jnp.exp(A * 0.25 + maskadd)
        P = E * (1.0 / (jnp.sum(E, axis=1, keepdims=True) + 1e-30))
        nn = lax.dot_general(P.astype(bf16), selV, NT_DIMS,
                             preferred_element_type=f32)
        nn_aug = jnp.concatenate([nn.astype(bf16), ones_col], axis=1)
        nnblks.append(lax.dot_general(nn_aug, place_ref[h], NN_DIMS,
                                      preferred_element_type=f32).astype(bf16))
    NNblk = jnp.concatenate(nnblks, axis=0)

    X1aug = lax.dot_general(S_cat, NNblk, TN_DIMS, preferred_element_type=f32)
    D = K_HEADS * D_HEAD
    X1 = X1aug[:, :D] * (1.0 / (X1aug[:, D:] + 1e-14))

    Hd = lax.dot_general(X1, w10t_ref[...], NN_DIMS, preferred_element_type=f32)
    Hd = jnp.maximum(Hd + b10_ref[...], 0.0)
    out = lax.dot_general(Hd, w11t_ref[...], NN_DIMS, preferred_element_type=f32)
    out_ref[0, 0] = out + b11_ref[...]


def kernel(X, STE, W7, b7, W8, b8, W9, b9, W10, b10, W11, b11, node_emb):
    B, T, N, D = X.shape
    f32 = jnp.float32

    def head_blocks(W):
        return (W[:, :D].reshape(K_HEADS, D_HEAD, D),
                W[:, D:].reshape(K_HEADS, D_HEAD, D))

    w7x, w7s = head_blocks(W7)
    w8x, w8s = head_blocks(W8)
    w9x, w9s = head_blocks(W9)
    WxT = jnp.concatenate(
        [jnp.concatenate([w7x, w8x], axis=1).reshape(NQK, D),
         w9x.reshape(K_HEADS * D_HEAD, D)], axis=0)
    WsT = jnp.concatenate(
        [jnp.concatenate([w7s, w8s], axis=1).reshape(NQK, D),
         w9s.reshape(K_HEADS * D_HEAD, D)], axis=0)
    bqkv = jnp.concatenate(
        [jnp.concatenate([b7.reshape(K_HEADS, D_HEAD),
                          b8.reshape(K_HEADS, D_HEAD)], axis=1).reshape(NQK),
         b9], axis=0).reshape(3 * D, 1)

    embpad = jnp.concatenate(
        [node_emb, jnp.zeros((MEM_PAD - MEM, QK), f32)], axis=0)
    ELqk = jnp.kron(jnp.eye(K_HEADS, dtype=f32), embpad)
    place = jnp.asarray(_PLACE, jnp.bfloat16)

    out = pl.pallas_call(
        _body,
        grid=(B, T),
        in_specs=[
            pl.BlockSpec((1, 1, N, D), lambda b, t: (b, t, 0, 0)),
            pl.BlockSpec((1, 1, N, D), lambda b, t: (b, t, 0, 0)),
            pl.BlockSpec((3 * D, D), lambda b, t: (0, 0)),
            pl.BlockSpec((3 * D, D), lambda b, t: (0, 0)),
            pl.BlockSpec((3 * D, 1), lambda b, t: (0, 0)),
            pl.BlockSpec((NQK, NQK), lambda b, t: (0, 0)),
            pl.BlockSpec((K_HEADS, QK, NQK), lambda b, t: (0, 0, 0)),
            pl.BlockSpec((D, D), lambda b, t: (0, 0)),
            pl.BlockSpec((1, D), lambda b, t: (0, 0)),
            pl.BlockSpec((D, D), lambda b, t: (0, 0)),
            pl.BlockSpec((1, D), lambda b, t: (0, 0)),
        ],
        out_specs=pl.BlockSpec((1, 1, N, D), lambda b, t: (b, t, 0, 0)),
        out_shape=jax.ShapeDtypeStruct((B, T, N, D), f32),
    )(X, STE, WxT, WsT, bqkv, ELqk, place, W10.T, b10.reshape(1, D),
      W11.T, b11.reshape(1, D))
    return out


# speedup vs baseline: 24.0297x; 1.0690x over previous
import numpy as np
import jax
import jax.numpy as jnp
from jax import lax
from jax.experimental import pallas as pl

K_HEADS = 8
D_HEAD = 16
TOPK = 10
MEM = 30
MEM_PAD = 32
NEG = -1e30
QK = 2 * D_HEAD
NQK = K_HEADS * QK
R = TOPK * MEM_PAD

NN_DIMS = (((1,), (0,)), ((), ()))
NT_DIMS = (((1,), (1,)), ((), ()))
TN_DIMS = (((0,), (0,)), ((), ()))


def _place_matrix():
    p = np.zeros((K_HEADS, QK, 2 * K_HEADS * D_HEAD), np.float32)
    for h in range(K_HEADS):
        for j in range(D_HEAD):
            p[h, j, D_HEAD * h + j] = 1.0
            p[h, D_HEAD + j, K_HEADS * D_HEAD + D_HEAD * h + j] = 1.0
    return p


_PLACE = _place_matrix()


def _body(x_ref, ste_ref, wxt_ref, wst_ref, bqkv_ref, elqk_ref, place_ref,
          w10t_ref, b10_ref, w11t_ref, b11_ref, out_ref):
    N = x_ref.shape[2]
    X = x_ref[0, 0]
    STE = ste_ref[0, 0]
    f32 = jnp.float32
    bf16 = jnp.bfloat16

    qkvt = lax.dot_general(wxt_ref[...], X, NT_DIMS, preferred_element_type=f32)
    qkvt += lax.dot_general(wst_ref[...], STE, NT_DIMS, preferred_element_type=f32)
    qkvt = jnp.maximum(qkvt + bqkv_ref[...], 0.0)

    L = lax.dot_general(elqk_ref[...], qkvt, NN_DIMS,
                        preferred_element_type=f32)

    padbump = jnp.where(
        lax.broadcasted_iota(jnp.int32, (MEM_PAD, 1), 0) < MEM, 0.0, 1e30)

    ri = lax.broadcasted_iota(jnp.int32, (R, R), 0) % MEM_PAD
    ci = lax.broadcasted_iota(jnp.int32, (R, R), 1) % MEM_PAD
    valid = (ri == ci) & (ri < MEM) & (ci < MEM)
    maskadd = jnp.where(valid, 0.0, NEG)
    ones_col = jnp.ones((R, D_HEAD), bf16)

    ohs = []
    for h in range(K_HEADS):
        Lh = L[QK * h: QK * h + MEM_PAD, :]
        for s in range(TOPK):
            m = jnp.max(Lh, axis=1, keepdims=True) + padbump
            eq = Lh >= m
            ohs.append(eq.astype(bf16))
            if s + 1 < TOPK:
                Lh = jnp.where(eq, NEG, Lh)
    S_cat = jnp.concatenate(ohs, axis=0)

    nnblks = []
    for h in range(K_HEADS):
        S_h = S_cat[R * h: R * (h + 1), :]
        qkv_b = qkvt[3 * D_HEAD * h: 3 * D_HEAD * (h + 1), :].astype(bf16)
        sel = lax.dot_general(qkv_b, S_h, NT_DIMS,
                              preferred_element_type=f32).astype(bf16)
        selV = sel[2 * D_HEAD:, :]

        A = lax.dot_general(sel[:D_HEAD, :], sel[D_HEAD:2 * D_HEAD, :],
                            TN_DIMS,
                            preferred_element_type=f32)
        E = jnp.exp(A * 0.25 + maskadd)
        P = E * (1.0 / (jnp.sum(E, axis=1, keepdims=True) + 1e-30))
        nn = lax.dot_general(P.astype(bf16), selV, NT_DIMS,
                             preferred_element_type=f32)
        nn_aug = jnp.concatenate([nn.astype(bf16), ones_col], axis=1)
        nnblks.append(lax.dot_general(nn_aug, place_ref[h], NN_DIMS,
                                      preferred_element_type=f32).astype(bf16))
    NNblk = jnp.concatenate(nnblks, axis=0)

    X1aug = lax.dot_general(S_cat, NNblk, TN_DIMS, preferred_element_type=f32)
    D = K_HEADS * D_HEAD
    X1 = X1aug[:, :D] * (1.0 / (X1aug[:, D:] + 1e-14))

    Hd = lax.dot_general(X1, w10t_ref[...], NN_DIMS, preferred_element_type=f32)
    Hd = jnp.maximum(Hd + b10_ref[...], 0.0)
    out = lax.dot_general(Hd, w11t_ref[...], NN_DIMS, preferred_element_type=f32)
    out_ref[0, 0] = out + b11_ref[...]


def kernel(X, STE, W7, b7, W8, b8, W9, b9, W10, b10, W11, b11, node_emb):
    B, T, N, D = X.shape
    f32 = jnp.float32

    def head_blocks(W):
        return (W[:, :D].reshape(K_HEADS, D_HEAD, D),
                W[:, D:].reshape(K_HEADS, D_HEAD, D))

    w7x, w7s = head_blocks(W7)
    w8x, w8s = head_blocks(W8)
    w9x, w9s = head_blocks(W9)
    WxT = jnp.concatenate([w7x, w8x, w9x], axis=1).reshape(3 * D, D)
    WsT = jnp.concatenate([w7s, w8s, w9s], axis=1).reshape(3 * D, D)
    bqkv = jnp.concatenate(
        [b7.reshape(K_HEADS, D_HEAD), b8.reshape(K_HEADS, D_HEAD),
         b9.reshape(K_HEADS, D_HEAD)], axis=1).reshape(3 * D, 1)

    embpad = jnp.concatenate(
        [node_emb, jnp.zeros((MEM, D_HEAD), f32)], axis=1)
    embpad = jnp.concatenate(
        [embpad, jnp.zeros((MEM_PAD - MEM, 3 * D_HEAD), f32)], axis=0)
    ELqk = jnp.kron(jnp.eye(K_HEADS, dtype=f32), embpad)
    place = jnp.asarray(_PLACE, jnp.bfloat16)

    out = pl.pallas_call(
        _body,
        grid=(B, T),
        in_specs=[
            pl.BlockSpec((1, 1, N, D), lambda b, t: (b, t, 0, 0)),
            pl.BlockSpec((1, 1, N, D), lambda b, t: (b, t, 0, 0)),
            pl.BlockSpec((3 * D, D), lambda b, t: (0, 0)),
            pl.BlockSpec((3 * D, D), lambda b, t: (0, 0)),
            pl.BlockSpec((3 * D, 1), lambda b, t: (0, 0)),
            pl.BlockSpec((NQK, 3 * D), lambda b, t: (0, 0)),
            pl.BlockSpec((K_HEADS, QK, NQK), lambda b, t: (0, 0, 0)),
            pl.BlockSpec((D, D), lambda b, t: (0, 0)),
            pl.BlockSpec((1, D), lambda b, t: (0, 0)),
            pl.BlockSpec((D, D), lambda b, t: (0, 0)),
            pl.BlockSpec((1, D), lambda b, t: (0, 0)),
        ],
        out_specs=pl.BlockSpec((1, 1, N, D), lambda b, t: (b, t, 0, 0)),
        out_shape=jax.ShapeDtypeStruct((B, T, N, D), f32),
    )(X, STE, WxT, WsT, bqkv, ELqk, place, W10.T, b10.reshape(1, D),
      W11.T, b11.reshape(1, D))
    return out


# speedup vs baseline: 31.2776x; 1.3016x over previous
import numpy as np
import jax
import jax.numpy as jnp
from jax import lax
from jax.experimental import pallas as pl

K_HEADS = 8
D_HEAD = 16
TOPK = 10
MEM = 30
MEM_PAD = 32
NEG = -1e30
QK = 2 * D_HEAD
NQK = K_HEADS * QK
R = TOPK * MEM_PAD

NN_DIMS = (((1,), (0,)), ((), ()))
NT_DIMS = (((1,), (1,)), ((), ()))
TN_DIMS = (((0,), (0,)), ((), ()))


def _place_matrix():
    p = np.zeros((K_HEADS, QK, 2 * K_HEADS * D_HEAD), np.float32)
    for h in range(K_HEADS):
        for j in range(D_HEAD):
            p[h, j, D_HEAD * h + j] = 1.0
            p[h, D_HEAD + j, K_HEADS * D_HEAD + D_HEAD * h + j] = 1.0
    return p


_PLACE = _place_matrix()


def _body(x_ref, ste_ref, wxt_ref, wst_ref, bqkv_ref, elqk_ref, place_ref,
          w10t_ref, b10_ref, w11t_ref, b11_ref, out_ref):
    N = x_ref.shape[2]
    X = x_ref[0, 0]
    STE = ste_ref[0, 0]
    f32 = jnp.float32
    bf16 = jnp.bfloat16

    qkvt = lax.dot_general(wxt_ref[...], X, NT_DIMS, preferred_element_type=f32)
    qkvt += lax.dot_general(wst_ref[...], STE, NT_DIMS, preferred_element_type=f32)
    qkvt = jnp.maximum(qkvt + bqkv_ref[...], 0.0)

    L = lax.dot_general(elqk_ref[...], qkvt, NN_DIMS,
                        preferred_element_type=f32)

    padbump = jnp.where(
        lax.broadcasted_iota(jnp.int32, (MEM_PAD, 1), 0) < MEM, 0.0, 1e30)

    ri = lax.broadcasted_iota(jnp.int32, (R, R), 0) % MEM_PAD
    ci = lax.broadcasted_iota(jnp.int32, (R, R), 1) % MEM_PAD
    valid = (ri == ci) & (ri < MEM) & (ci < MEM)
    maskadd = jnp.where(valid, 0.0, NEG)
    ones_col = jnp.ones((R, D_HEAD), bf16)

    ohs = []
    for h in range(K_HEADS):
        Lh = L[QK * h: QK * h + MEM_PAD, :]
        for s in range(TOPK):
            m = jnp.max(Lh, axis=1, keepdims=True) + padbump
            eq = Lh >= m
            ohs.append(eq.astype(bf16))
            if s + 1 < TOPK:
                Lh = jnp.where(eq, NEG, Lh)
    S_cat = jnp.concatenate(ohs, axis=0)

    sels = []
    for h in range(K_HEADS):
        S_h = S_cat[R * h: R * (h + 1), :]
        qkv_b = qkvt[3 * D_HEAD * h: 3 * D_HEAD * (h + 1), :].astype(bf16)
        sels.append(lax.dot_general(qkv_b, S_h, NT_DIMS,
                                    preferred_element_type=f32).astype(bf16))
    As = [lax.dot_general(sel[:D_HEAD, :], sel[D_HEAD:2 * D_HEAD, :], TN_DIMS,
                          preferred_element_type=f32) for sel in sels]
    Es = [jnp.exp(A * 0.25 + maskadd) for A in As]
    Ps = [(E * (1.0 / (jnp.sum(E, axis=1, keepdims=True) + 1e-30))).astype(bf16)
          for E in Es]
    nns = [lax.dot_general(Ps[h], sels[h][2 * D_HEAD:, :], NT_DIMS,
                           preferred_element_type=f32) for h in range(K_HEADS)]
    nnblks = [lax.dot_general(
        jnp.concatenate([nns[h].astype(bf16), ones_col], axis=1),
        place_ref[h], NN_DIMS,
        preferred_element_type=f32).astype(bf16) for h in range(K_HEADS)]
    NNblk = jnp.concatenate(nnblks, axis=0)

    X1aug = lax.dot_general(S_cat, NNblk, TN_DIMS, preferred_element_type=f32)
    D = K_HEADS * D_HEAD
    X1 = X1aug[:, :D] * (1.0 / (X1aug[:, D:] + 1e-14))

    Hd = lax.dot_general(X1, w10t_ref[...], NN_DIMS, preferred_element_type=f32)
    Hd = jnp.maximum(Hd + b10_ref[...], 0.0)
    out = lax.dot_general(Hd, w11t_ref[...], NN_DIMS, preferred_element_type=f32)
    out_ref[0, 0] = out + b11_ref[...]


def kernel(X, STE, W7, b7, W8, b8, W9, b9, W10, b10, W11, b11, node_emb):
    B, T, N, D = X.shape
    f32 = jnp.float32

    def head_blocks(W):
        return (W[:, :D].reshape(K_HEADS, D_HEAD, D),
                W[:, D:].reshape(K_HEADS, D_HEAD, D))

    w7x, w7s = head_blocks(W7)
    w8x, w8s = head_blocks(W8)
    w9x, w9s = head_blocks(W9)
    WxT = jnp.concatenate([w7x, w8x, w9x], axis=1).reshape(3 * D, D)
    WsT = jnp.concatenate([w7s, w8s, w9s], axis=1).reshape(3 * D, D)
    bqkv = jnp.concatenate(
        [b7.reshape(K_HEADS, D_HEAD), b8.reshape(K_HEADS, D_HEAD),
         b9.reshape(K_HEADS, D_HEAD)], axis=1).reshape(3 * D, 1)

    embpad = jnp.concatenate(
        [node_emb, jnp.zeros((MEM, D_HEAD), f32)], axis=1)
    embpad = jnp.concatenate(
        [embpad, jnp.zeros((MEM_PAD - MEM, 3 * D_HEAD), f32)], axis=0)
    ELqk = jnp.kron(jnp.eye(K_HEADS, dtype=f32), embpad)
    place = jnp.asarray(_PLACE, jnp.bfloat16)

    out = pl.pallas_call(
        _body,
        grid=(B, T),
        in_specs=[
            pl.BlockSpec((1, 1, N, D), lambda b, t: (b, t, 0, 0)),
            pl.BlockSpec((1, 1, N, D), lambda b, t: (b, t, 0, 0)),
            pl.BlockSpec((3 * D, D), lambda b, t: (0, 0)),
            pl.BlockSpec((3 * D, D), lambda b, t: (0, 0)),
            pl.BlockSpec((3 * D, 1), lambda b, t: (0, 0)),
            pl.BlockSpec((NQK, 3 * D), lambda b, t: (0, 0)),
            pl.BlockSpec((K_HEADS, QK, NQK), lambda b, t: (0, 0, 0)),
            pl.BlockSpec((D, D), lambda b, t: (0, 0)),
            pl.BlockSpec((1, D), lambda b, t: (0, 0)),
            pl.BlockSpec((D, D), lambda b, t: (0, 0)),
            pl.BlockSpec((1, D), lambda b, t: (0, 0)),
        ],
        out_specs=pl.BlockSpec((1, 1, N, D), lambda b, t: (b, t, 0, 0)),
        out_shape=jax.ShapeDtypeStruct((B, T, N, D), f32),
    )(X, STE, WxT, WsT, bqkv, ELqk, place, W10.T, b10.reshape(1, D),
      W11.T, b11.reshape(1, D))
    return out


# speedup vs baseline: 31.3130x; 1.0011x over previous
import numpy as np
import jax
import jax.numpy as jnp
from jax import lax
from jax.experimental import pallas as pl

K_HEADS = 8
D_HEAD = 16
TOPK = 10
MEM = 30
MEM_PAD = 32
NEG = -1e30
QK = 2 * D_HEAD
NQK = K_HEADS * QK
R = TOPK * MEM_PAD

NN_DIMS = (((1,), (0,)), ((), ()))
NT_DIMS = (((1,), (1,)), ((), ()))
TN_DIMS = (((0,), (0,)), ((), ()))


def _place_matrix():
    p = np.zeros((K_HEADS, QK, 2 * K_HEADS * D_HEAD), np.float32)
    for h in range(K_HEADS):
        for j in range(D_HEAD):
            p[h, j, D_HEAD * h + j] = 1.0
            p[h, D_HEAD + j, K_HEADS * D_HEAD + D_HEAD * h + j] = 1.0
    return p


_PLACE = _place_matrix()


def _body(x_ref, ste_ref, wxt_ref, wst_ref, bqkv_ref, elqk_ref, place_ref,
          w10t_ref, b10_ref, w11t_ref, b11_ref, out_ref):
    N = x_ref.shape[2]
    X = x_ref[0, 0]
    STE = ste_ref[0, 0]
    f32 = jnp.float32
    bf16 = jnp.bfloat16

    qkvt = lax.dot_general(wxt_ref[...], X, NT_DIMS, preferred_element_type=f32)
    qkvt += lax.dot_general(wst_ref[...], STE, NT_DIMS, preferred_element_type=f32)
    qkvt = jnp.maximum(qkvt + bqkv_ref[...], 0.0)

    L = lax.dot_general(elqk_ref[...], qkvt, NN_DIMS,
                        preferred_element_type=f32)

    padbump = jnp.where(
        lax.broadcasted_iota(jnp.int32, (MEM_PAD, 1), 0) < MEM, 0.0, 1e30)

    ri = lax.broadcasted_iota(jnp.int32, (R, R), 0) % MEM_PAD
    ci = lax.broadcasted_iota(jnp.int32, (R, R), 1) % MEM_PAD
    valid = (ri == ci) & (ri < MEM) & (ci < MEM)
    maskadd = jnp.where(valid, 0.0, NEG)
    ones_col = jnp.ones((R, D_HEAD), bf16)

    Lhs = [L[QK * h: QK * h + MEM_PAD, :] for h in range(K_HEADS)]
    ohs = [[None] * TOPK for _ in range(K_HEADS)]
    for s in range(TOPK):
        ms = [jnp.max(Lh, axis=1, keepdims=True) + padbump for Lh in Lhs]
        eqs = [Lhs[h] >= ms[h] for h in range(K_HEADS)]
        for h in range(K_HEADS):
            ohs[h][s] = eqs[h].astype(bf16)
        if s + 1 < TOPK:
            Lhs = [jnp.where(eqs[h], NEG, Lhs[h]) for h in range(K_HEADS)]
    S_cat = jnp.concatenate(
        [oh for head_ohs in ohs for oh in head_ohs], axis=0)

    sels = []
    for h in range(K_HEADS):
        S_h = S_cat[R * h: R * (h + 1), :]
        qkv_b = qkvt[3 * D_HEAD * h: 3 * D_HEAD * (h + 1), :].astype(bf16)
        sels.append(lax.dot_general(qkv_b, S_h, NT_DIMS,
                                    preferred_element_type=f32).astype(bf16))
    As = [lax.dot_general(sel[:D_HEAD, :], sel[D_HEAD:2 * D_HEAD, :], TN_DIMS,
                          preferred_element_type=f32) for sel in sels]
    Es = [jnp.exp(A * 0.25 + maskadd) for A in As]
    Ps = [(E * (1.0 / (jnp.sum(E, axis=1, keepdims=True) + 1e-30))).astype(bf16)
          for E in Es]
    nns = [lax.dot_general(Ps[h], sels[h][2 * D_HEAD:, :], NT_DIMS,
                           preferred_element_type=f32) for h in range(K_HEADS)]
    nnblks = [lax.dot_general(
        jnp.concatenate([nns[h].astype(bf16), ones_col], axis=1),
        place_ref[h], NN_DIMS,
        preferred_element_type=f32).astype(bf16) for h in range(K_HEADS)]
    NNblk = jnp.concatenate(nnblks, axis=0)

    X1aug = lax.dot_general(S_cat, NNblk, TN_DIMS, preferred_element_type=f32)
    D = K_HEADS * D_HEAD
    X1 = X1aug[:, :D] * (1.0 / (X1aug[:, D:] + 1e-14))

    Hd = lax.dot_general(X1, w10t_ref[...], NN_DIMS, preferred_element_type=f32)
    Hd = jnp.maximum(Hd + b10_ref[...], 0.0)
    out = lax.dot_general(Hd, w11t_ref[...], NN_DIMS, preferred_element_type=f32)
    out_ref[0, 0] = out + b11_ref[...]


def kernel(X, STE, W7, b7, W8, b8, W9, b9, W10, b10, W11, b11, node_emb):
    B, T, N, D = X.shape
    f32 = jnp.float32

    def head_blocks(W):
        return (W[:, :D].reshape(K_HEADS, D_HEAD, D),
                W[:, D:].reshape(K_HEADS, D_HEAD, D))

    w7x, w7s = head_blocks(W7)
    w8x, w8s = head_blocks(W8)
    w9x, w9s = head_blocks(W9)
    WxT = jnp.concatenate([w7x, w8x, w9x], axis=1).reshape(3 * D, D)
    WsT = jnp.concatenate([w7s, w8s, w9s], axis=1).reshape(3 * D, D)
    bqkv = jnp.concatenate(
        [b7.reshape(K_HEADS, D_HEAD), b8.reshape(K_HEADS, D_HEAD),
         b9.reshape(K_HEADS, D_HEAD)], axis=1).reshape(3 * D, 1)

    embpad = jnp.concatenate(
        [node_emb, jnp.zeros((MEM, D_HEAD), f32)], axis=1)
    embpad = jnp.concatenate(
        [embpad, jnp.zeros((MEM_PAD - MEM, 3 * D_HEAD), f32)], axis=0)
    ELqk = jnp.kron(jnp.eye(K_HEADS, dtype=f32), embpad)
    place = jnp.asarray(_PLACE, jnp.bfloat16)

    out = pl.pallas_call(
        _body,
        grid=(B, T),
        in_specs=[
            pl.BlockSpec((1, 1, N, D), lambda b, t: (b, t, 0, 0)),
            pl.BlockSpec((1, 1, N, D), lambda b, t: (b, t, 0, 0)),
            pl.BlockSpec((3 * D, D), lambda b, t: (0, 0)),
            pl.BlockSpec((3 * D, D), lambda b, t: (0, 0)),
            pl.BlockSpec((3 * D, 1), lambda b, t: (0, 0)),
            pl.BlockSpec((NQK, 3 * D), lambda b, t: (0, 0)),
            pl.BlockSpec((K_HEADS, QK, NQK), lambda b, t: (0, 0, 0)),
            pl.BlockSpec((D, D), lambda b, t: (0, 0)),
            pl.BlockSpec((1, D), lambda b, t: (0, 0)),
            pl.BlockSpec((D, D), lambda b, t: (0, 0)),
            pl.BlockSpec((1, D), lambda b, t: (0, 0)),
        ],
        out_specs=pl.BlockSpec((1, 1, N, D), lambda b, t: (b, t, 0, 0)),
        out_shape=jax.ShapeDtypeStruct((B, T, N, D), f32),
    )(X, STE, WxT, WsT, bqkv, ELqk, place, W10.T, b10.reshape(1, D),
      W11.T, b11.reshape(1, D))
    return out
